# baseline (speedup 1.0000x reference)
import functools

import jax
import jax.numpy as jnp
from jax import lax
from jax.experimental import pallas as pl
from jax.experimental.pallas import tpu as pltpu
from jax.experimental.pallas import tpu_sc as plsc

N = 100000
E = 3200000
C = 16
NUM_CLASSES = 10
NC = 2
NS = 16
NW = NC * NS
K_AP = 10
K_PR = 10
ALPHA = 0.1
WMIX = 0.9

N_D = 102400
CH = 1024
NCHUNK = 98
NCHUNK_F = 2 * NCHUNK
EPW = CH * NCHUNK
E_HALF = EPW * NS
E_PAD = 2 * E_HALF
IDXROWS = E_PAD // 128
RPW_D = N_D // NW
DCHUNKS = ((0, 512), (512, 512), (1024, 512), (1536, 512),
           (2048, 512), (2560, 512), (3072, 128))
N_ACC = 100096
SL = N_ACC // NS
ZCH = SL // 16

_mesh = plsc.VectorSubcoreMesh(core_axis_name="c", subcore_axis_name="s")
_mesh1 = plsc.VectorSubcoreMesh(core_axis_name="c", subcore_axis_name="s",
                                num_cores=1)
_cparams = pltpu.CompilerParams(use_tc_tiling_on_sc=False,
                                needs_layout_passes=False)


def _wid():
    return lax.axis_index("c") * NS + lax.axis_index("s")


def _rsqrt(x):
    i = lax.bitcast_convert_type(x, jnp.int32)
    i = jnp.int32(0x5F3759DF) - lax.shift_right_arithmetic(i, 1)
    y = lax.bitcast_convert_type(i, jnp.float32)
    for _ in range(4):
        y = y * (1.5 - 0.5 * x * y * y)
    return y


def _zero_acc(acc, zbuf):
    zrow = jnp.zeros((C,), jnp.float32)

    @pl.loop(0, ZCH)
    def _(i):
        zbuf[i] = zrow

    sid = lax.axis_index("s")
    for j in range(16):
        pltpu.sync_copy(zbuf, acc.at[pl.ds(sid * SL + j * ZCH, ZCH), :])


def _dump_acc(acc, part_hbm):
    sid = lax.axis_index("s")
    pltpu.sync_copy(acc.at[pl.ds(sid * SL, SL), :],
                    part_hbm.at[pl.ds(sid * SL, SL), :])


@functools.partial(
    pl.kernel,
    out_type=jax.ShapeDtypeStruct((N_D, C), jnp.float32),
    mesh=_mesh1,
    compiler_params=_cparams,
    scratch_types=[
        pltpu.VMEM((8, 128), jnp.int32),
        pltpu.VMEM((8, 128), jnp.int32),
        pltpu.VMEM((CH, C), jnp.float32),
        pltpu.VMEM((ZCH, C), jnp.float32),
        pltpu.VMEM_SHARED((N_ACC, C), jnp.float32),
        pltpu.SemaphoreType.DMA,
        pltpu.SemaphoreType.DMA,
        pltpu.SemaphoreType.DMA,
    ],
)
def _scatter_k(u_hbm, src_hbm, dst_hbm, part_hbm,
               sbuf, dbuf, rows, zbuf, acc, sem_i, sem_g, sem_s):
    wid = lax.axis_index("s")
    _zero_acc(acc, zbuf)
    plsc.subcore_barrier()

    @pl.loop(0, NCHUNK)
    def _(ch):
        base = wid * (EPW // 128) + ch * 8
        c1 = pltpu.async_copy(src_hbm.at[pl.ds(base, 8), :], sbuf, sem_i)
        c2 = pltpu.async_copy(dst_hbm.at[pl.ds(base, 8), :], dbuf, sem_i)
        c1.wait()
        c2.wait()
        gets = [
            pltpu.async_copy(u_hbm.at[sbuf.at[i]],
                             rows.at[pl.ds(i * 128, 128), :], sem_g)
            for i in range(8)
        ]
        for g in gets:
            g.wait()
        puts = [
            pltpu.async_copy(rows.at[pl.ds(i * 128, 128), :],
                             acc.at[dbuf.at[i]], sem_s, add=True)
            for i in range(8)
        ]
        for p in puts:
            p.wait()

    plsc.subcore_barrier()
    _dump_acc(acc, part_hbm)


@functools.partial(
    pl.kernel,
    out_type=(
        jax.ShapeDtypeStruct((N_D, C), jnp.float32),
        jax.ShapeDtypeStruct((NS, C), jnp.float32),
    ),
    mesh=_mesh1,
    compiler_params=_cparams,
    scratch_types=[
        pltpu.VMEM((8, 128), jnp.int32),
        pltpu.VMEM((128, C), jnp.float32),
        pltpu.VMEM((ZCH, C), jnp.float32),
        pltpu.VMEM((528,), jnp.int32),
        pltpu.VMEM((528,), jnp.float32),
        pltpu.VMEM((C,), jnp.float32),
        pltpu.VMEM_SHARED((N_ACC, C), jnp.float32),
        pltpu.SemaphoreType.DMA,
        pltpu.SemaphoreType.DMA,
    ],
)
def _count_k(dst_hbm, y_hbm, tm_hbm, deg_hbm, lc_hbm,
             dbuf, ones, zbuf, ybuf, tbuf, lcbuf, acc, sem_i, sem_s):
    wid = lax.axis_index("s")
    one_row = jnp.ones((C,), jnp.float32)

    @pl.loop(0, 128)
    def _(i):
        ones[i] = one_row

    _zero_acc(acc, zbuf)
    plsc.subcore_barrier()

    @pl.loop(0, NCHUNK_F)
    def _(ch):
        base = wid * (2 * EPW // 128) + ch * 8
        pltpu.async_copy(dst_hbm.at[pl.ds(base, 8), :], dbuf, sem_i).wait()
        puts = [
            pltpu.async_copy(ones, acc.at[dbuf.at[i]], sem_s, add=True)
            for i in range(8)
        ]
        for p in puts:
            p.wait()

    plsc.subcore_barrier()
    _dump_acc(acc, deg_hbm)

    base = wid * (N_D // NS)
    lc = jnp.zeros((C,), jnp.float32)
    lanes = lax.iota(jnp.int32, C)
    count_chunks = tuple((o * 512, 512) for o in range(12)) + ((6144, 256),)
    for off, rows in count_chunks:
        c1 = pltpu.async_copy(y_hbm.at[pl.ds(base + off, rows)],
                              ybuf.at[pl.ds(0, rows)], sem_i)
        c2 = pltpu.async_copy(tm_hbm.at[pl.ds(base + off, rows)],
                              tbuf.at[pl.ds(0, rows)], sem_i)
        c1.wait()
        c2.wait()

        def body(i, carry):
            yv = ybuf[pl.ds(i, 16)][0]
            tv = tbuf[pl.ds(i, 16)][0]
            return carry + jnp.where(lanes == yv, tv, 0.0)

        lc = pl.loop(0, rows, init_carry=lc)(body)
    lcbuf[...] = lc
    pltpu.sync_copy(lcbuf, lc_hbm.at[wid])


def _make_dense(teleport):
    if teleport is None:
        out_type = (
            jax.ShapeDtypeStruct((N_D, C), jnp.float32),
            jax.ShapeDtypeStruct((N_D, C), jnp.float32),
            jax.ShapeDtypeStruct((N_D, C), jnp.float32),
        )
    else:
        out_type = (
            jax.ShapeDtypeStruct((N_D, C), jnp.float32),
            jax.ShapeDtypeStruct((N_D, C), jnp.float32),
        )

    @functools.partial(
        pl.kernel,
        out_type=out_type,
        mesh=_mesh,
        compiler_params=_cparams,
        scratch_types=[
            pltpu.VMEM((512, C), jnp.float32),
            pltpu.VMEM((512, C), jnp.float32),
            pltpu.VMEM((512, C), jnp.float32),
            pltpu.VMEM((512, C), jnp.float32),
            pltpu.VMEM((512, C), jnp.float32),
            pltpu.VMEM((512, C), jnp.float32),
            pltpu.VMEM((528,), jnp.int32),
            pltpu.VMEM((528,), jnp.float32),
            pltpu.VMEM((C,), jnp.float32),
            pltpu.SemaphoreType.DMA,
            pltpu.SemaphoreType.DMA,
        ],
    )
    def dense_k(*args):
        if teleport is None:
            (deg_hbm, y_hbm, tm_hbm, lc_hbm, dinv_hbm, puc_hbm, u0_hbm,
             p0, p1, _hh, _pp, outa, outb, ybuf, tbuf, lcv,
             sem_i, sem_o) = args
        elif teleport:
            (parta_hbm, partb_hbm, h_hbm, dinv_hbm, puc_hbm, hn_hbm, un_hbm,
             p0, p1, hh, pp, outa, outb, _yb, _tb, _lcv,
             sem_i, sem_o) = args
        else:
            (parta_hbm, partb_hbm, h_hbm, dinv_hbm, hn_hbm, un_hbm,
             p0, p1, hh, pp, outa, outb, _yb, _tb, _lcv,
             sem_i, sem_o) = args
        wid = _wid()
        base = wid * RPW_D
        lanes = lax.iota(jnp.int32, C)
        if teleport is None:
            pltpu.sync_copy(lc_hbm, lcv)
            lcm = jnp.maximum(lcv[...], 1.0)
        for off, rows in DCHUNKS:
            r0 = base + off
            cps = []
            if teleport is None:
                cps.append(pltpu.async_copy(
                    deg_hbm.at[pl.ds(r0, rows), :],
                    p0.at[pl.ds(0, rows), :], sem_i))
                cps.append(pltpu.async_copy(
                    y_hbm.at[pl.ds(r0, rows)], ybuf.at[pl.ds(0, rows)],
                    sem_i))
                cps.append(pltpu.async_copy(
                    tm_hbm.at[pl.ds(r0, rows)], tbuf.at[pl.ds(0, rows)],
                    sem_i))
            else:
                cps.append(pltpu.async_copy(
                    parta_hbm.at[pl.ds(r0, rows), :],
                    p0.at[pl.ds(0, rows), :], sem_i))
                cps.append(pltpu.async_copy(
                    partb_hbm.at[pl.ds(r0, rows), :],
                    p1.at[pl.ds(0, rows), :], sem_i))
                cps.append(pltpu.async_copy(
                    h_hbm.at[pl.ds(r0, rows), :],
                    hh.at[pl.ds(0, rows), :], sem_i))
                cps.append(pltpu.async_copy(
                    dinv_hbm.at[pl.ds(r0, rows), :],
                    pp.at[pl.ds(0, rows), :], sem_i))
                if teleport:
                    cps.append(pltpu.async_copy(
                        puc_hbm.at[pl.ds(r0, rows), :],
                        outb.at[pl.ds(0, rows), :], sem_i))
            for cp in cps:
                cp.wait()

            if teleport is None:
                @pl.loop(0, rows)
                def _(i):
                    deg = p0[i] + 1.0
                    d = _rsqrt(jnp.maximum(deg, 1.0))
                    yv = ybuf[pl.ds(i, 16)][0]
                    tv = tbuf[pl.ds(i, 16)][0]
                    puc = jnp.where(lanes == yv, tv, 0.0) / lcm
                    p0[i] = d
                    p1[i] = puc
                    outa[i] = d * puc
                ocps = [
                    pltpu.async_copy(p0.at[pl.ds(0, rows), :],
                                     dinv_hbm.at[pl.ds(r0, rows), :], sem_o),
                    pltpu.async_copy(p1.at[pl.ds(0, rows), :],
                                     puc_hbm.at[pl.ds(r0, rows), :], sem_o),
                    pltpu.async_copy(outa.at[pl.ds(0, rows), :],
                                     u0_hbm.at[pl.ds(r0, rows), :], sem_o),
                ]
            else:
                @pl.loop(0, rows)
                def _(i):
                    d = pp[i]
                    prop = d * (p0[i] + p1[i]) + d * d * hh[i]
                    if teleport:
                        hn = (1.0 - ALPHA) * prop + ALPHA * outb[i]
                    else:
                        hn = prop
                    outa[i] = hn
                    hh[i] = d * hn
                ocps = [
                    pltpu.async_copy(outa.at[pl.ds(0, rows), :],
                                     hn_hbm.at[pl.ds(r0, rows), :], sem_o),
                    pltpu.async_copy(hh.at[pl.ds(0, rows), :],
                                     un_hbm.at[pl.ds(r0, rows), :], sem_o),
                ]
            for cp in ocps:
                cp.wait()

    return dense_k


_init_k = _make_dense(None)
_blend_ap = _make_dense(True)
_blend_pr = _make_dense(False)


@functools.partial(
    pl.kernel,
    out_type=(
        jax.ShapeDtypeStruct((N_D, C), jnp.float32),
        jax.ShapeDtypeStruct((N_D // 16, C), jnp.float32),
    ),
    mesh=_mesh,
    compiler_params=_cparams,
    scratch_types=[
        pltpu.VMEM((512, C), jnp.float32),
        pltpu.VMEM((512, C), jnp.float32),
        pltpu.VMEM((512, C), jnp.float32),
        pltpu.VMEM((32, C), jnp.float32),
        pltpu.VMEM((C,), jnp.float32),
        pltpu.SemaphoreType.DMA,
        pltpu.SemaphoreType.DMA,
    ],
)
def _final_k(h_hbm, g_hbm, lc_hbm, out_hbm, pu_hbm,
             hh, gg, oo, pub, lcv, sem_i, sem_o):
    wid = _wid()
    base = wid * RPW_D
    lanes = lax.iota(jnp.int32, C)
    pltpu.sync_copy(lc_hbm, lcv)
    lc = lcv[...]
    pc = lc / jnp.sum(lc)
    for off, rows in DCHUNKS:
        r0 = base + off
        c1 = pltpu.async_copy(h_hbm.at[pl.ds(r0, rows), :],
                              hh.at[pl.ds(0, rows), :], sem_i)
        c2 = pltpu.async_copy(g_hbm.at[pl.ds(r0, rows), :],
                              gg.at[pl.ds(0, rows), :], sem_i)
        c1.wait()
        c2.wait()

        @pl.loop(0, rows // 16)
        def _(gr):
            pu = jnp.zeros((C,), jnp.float32)
            for l in range(16):
                i = gr * 16 + l
                o = (WMIX * hh[i] + (1.0 - WMIX) * gg[i]) * pc
                oo[i] = o
                pu = jnp.where(lanes == l, jnp.sum(o), pu)
            pub[gr] = pu
        ocps = [
            pltpu.async_copy(oo.at[pl.ds(0, rows), :],
                             out_hbm.at[pl.ds(r0, rows), :], sem_o),
            pltpu.async_copy(pub.at[pl.ds(0, rows // 16), :],
                             pu_hbm.at[pl.ds(r0 // 16, rows // 16), :],
                             sem_o),
        ]
        for cp in ocps:
            cp.wait()


@jax.jit
def kernel(x, y, train_mask, edge_index):
    del x
    pad_e = E_PAD - E
    src = jnp.concatenate(
        [edge_index[0], jnp.full((pad_e,), N, jnp.int32)]).reshape(
            IDXROWS, 128)
    dst = jnp.concatenate(
        [edge_index[1], jnp.full((pad_e,), N, jnp.int32)]).reshape(
            IDXROWS, 128)
    pad_n = N_D - N
    y_p = jnp.concatenate([y, jnp.zeros((pad_n,), jnp.int32)])
    tm_p = jnp.concatenate(
        [train_mask.astype(jnp.float32), jnp.zeros((pad_n,), jnp.float32)])

    deg_parts, lc_parts = _count_k(dst, y_p, tm_p)
    lc = lc_parts.sum(0)
    dinv, puc, u0 = _init_k(deg_parts, y_p, tm_p, lc)

    half = E_HALF // 128
    src_a, src_b = src[:half], src[half:]
    dst_a, dst_b = dst[:half], dst[half:]

    h, u = puc, u0
    for _ in range(K_AP):
        part_a = _scatter_k(u, src_a, dst_a)
        part_b = _scatter_k(u, src_b, dst_b)
        h, u = _blend_ap(part_a, part_b, h, dinv, puc)
    g, u = puc, u0
    for _ in range(K_PR):
        part_a = _scatter_k(u, src_a, dst_a)
        part_b = _scatter_k(u, src_b, dst_b)
        g, u = _blend_pr(part_a, part_b, g, dinv)

    out16, pu = _final_k(h, g, lc)
    out = out16[:N, :NUM_CLASSES]
    p_u = pu.reshape(-1)[:N]
    p_c = lc[:NUM_CLASSES] / lc.sum()
    return (out, p_u, p_c)


# speedup vs baseline: 1.1962x; 1.1962x over previous
import functools

import jax
import jax.numpy as jnp
from jax import lax
from jax.experimental import pallas as pl
from jax.experimental.pallas import tpu as pltpu
from jax.experimental.pallas import tpu_sc as plsc

N = 100000
E = 3200000
C = 16
NUM_CLASSES = 10
NC = 2
NS = 16
NW = NC * NS
K_AP = 10
K_PR = 10
ALPHA = 0.1
WMIX = 0.9

N_D = 102400
CH = 512
CR = CH // 128
NCHUNK = 392
EPW = CH * NCHUNK
NCHUNK_F = EPW // 1024
E_PAD = EPW * NS
IDXROWS = E_PAD // 128
RPW_D = N_D // NW
DCHUNKS = ((0, 512), (512, 512), (1024, 512), (1536, 512),
           (2048, 512), (2560, 512), (3072, 128))
N_ACC = 100096
SL = N_ACC // NS
ZCH = SL // 16

_mesh = plsc.VectorSubcoreMesh(core_axis_name="c", subcore_axis_name="s")
_mesh1 = plsc.VectorSubcoreMesh(core_axis_name="c", subcore_axis_name="s",
                                num_cores=1)
_cparams = pltpu.CompilerParams(use_tc_tiling_on_sc=False,
                                needs_layout_passes=False)


def _wid():
    return lax.axis_index("c") * NS + lax.axis_index("s")


def _rsqrt(x):
    i = lax.bitcast_convert_type(x, jnp.int32)
    i = jnp.int32(0x5F3759DF) - lax.shift_right_arithmetic(i, 1)
    y = lax.bitcast_convert_type(i, jnp.float32)
    for _ in range(4):
        y = y * (1.5 - 0.5 * x * y * y)
    return y


def _zero_acc(acc, zbuf):
    zrow = jnp.zeros((C,), jnp.float32)

    @pl.loop(0, ZCH)
    def _(i):
        zbuf[i] = zrow

    sid = lax.axis_index("s")
    for j in range(16):
        pltpu.sync_copy(zbuf.at[pl.ds(0, ZCH), :],
                        acc.at[pl.ds(sid * SL + j * ZCH, ZCH), :])


def _dump_acc(acc, part_hbm):
    sid = lax.axis_index("s")
    pltpu.sync_copy(acc.at[pl.ds(sid * SL, SL), :],
                    part_hbm.at[pl.ds(sid * SL, SL), :])


@functools.partial(
    pl.kernel,
    out_type=jax.ShapeDtypeStruct((N_D, C), jnp.float32),
    mesh=_mesh1,
    compiler_params=_cparams,
    scratch_types=[
        pltpu.VMEM((CR, 128), jnp.int32),
        pltpu.VMEM((CR, 128), jnp.int32),
        pltpu.VMEM((CR, 128), jnp.int32),
        pltpu.VMEM((CR, 128), jnp.int32),
        pltpu.VMEM((CH, C), jnp.float32),
        pltpu.VMEM((CH, C), jnp.float32),
        pltpu.VMEM_SHARED((N_ACC, C), jnp.float32),
        pltpu.SemaphoreType.DMA,
        pltpu.SemaphoreType.DMA,
        pltpu.SemaphoreType.DMA,
    ],
)
def _scatter_k(u_hbm, src_hbm, dst_hbm, part_hbm,
               sb_a, db_a, sb_b, db_b, rows_a, rows_b, acc,
               sem_i, sem_g, sem_s):
    wid = lax.axis_index("s")
    base0 = wid * (EPW // 128)

    def load_idx(ch, sb, db):
        c1 = pltpu.async_copy(src_hbm.at[pl.ds(base0 + ch * CR, CR), :],
                              sb, sem_i)
        c2 = pltpu.async_copy(dst_hbm.at[pl.ds(base0 + ch * CR, CR), :],
                              db, sem_i)
        c1.wait()
        c2.wait()

    def gathers(sb, rows):
        return [pltpu.make_async_copy(u_hbm.at[sb.at[i]],
                                      rows.at[pl.ds(i * 128, 128), :], sem_g)
                for i in range(CR)]

    def scatters(db, rows):
        return [pltpu.make_async_copy(rows.at[pl.ds(i * 128, 128), :],
                                      acc.at[db.at[i]], sem_s)
                for i in range(CR)]

    _zero_acc(acc, rows_a)
    zrow = jnp.zeros((C,), jnp.float32)

    @pl.loop(0, CH)
    def _(i):
        rows_b[i] = zrow

    plsc.subcore_barrier()

    load_idx(0, sb_b, db_b)
    for p in scatters(db_b, rows_b):
        p.start(add=True)
    load_idx(0, sb_a, db_a)
    for g in gathers(sb_a, rows_a):
        g.start()

    @pl.loop(0, NCHUNK, step=2)
    def _(ch):
        for p in scatters(db_b, rows_b):
            p.wait()
        load_idx(ch + 1, sb_b, db_b)
        for g in gathers(sb_a, rows_a):
            g.wait()
        for p in scatters(db_a, rows_a):
            p.start(add=True)
        for g in gathers(sb_b, rows_b):
            g.start()
        for p in scatters(db_a, rows_a):
            p.wait()
        c2 = jnp.minimum(ch + 2, NCHUNK - 1)
        load_idx(c2, sb_a, db_a)
        for g in gathers(sb_b, rows_b):
            g.wait()
        for p in scatters(db_b, rows_b):
            p.start(add=True)
        for g in gathers(sb_a, rows_a):
            g.start()

    for g in gathers(sb_a, rows_a):
        g.wait()
    for p in scatters(db_b, rows_b):
        p.wait()

    plsc.subcore_barrier()
    _dump_acc(acc, part_hbm)


@functools.partial(
    pl.kernel,
    out_type=(
        jax.ShapeDtypeStruct((N_D, C), jnp.float32),
        jax.ShapeDtypeStruct((NS, C), jnp.float32),
    ),
    mesh=_mesh1,
    compiler_params=_cparams,
    scratch_types=[
        pltpu.VMEM((8, 128), jnp.int32),
        pltpu.VMEM((128, C), jnp.float32),
        pltpu.VMEM((ZCH, C), jnp.float32),
        pltpu.VMEM((528,), jnp.int32),
        pltpu.VMEM((528,), jnp.float32),
        pltpu.VMEM((C,), jnp.float32),
        pltpu.VMEM_SHARED((N_ACC, C), jnp.float32),
        pltpu.SemaphoreType.DMA,
        pltpu.SemaphoreType.DMA,
    ],
)
def _count_k(dst_hbm, y_hbm, tm_hbm, deg_hbm, lc_hbm,
             dbuf, ones, zbuf, ybuf, tbuf, lcbuf, acc, sem_i, sem_s):
    wid = lax.axis_index("s")
    one_row = jnp.ones((C,), jnp.float32)

    @pl.loop(0, 128)
    def _(i):
        ones[i] = one_row

    _zero_acc(acc, zbuf)
    plsc.subcore_barrier()

    @pl.loop(0, NCHUNK_F)
    def _(ch):
        base = wid * (EPW // 128) + ch * 8
        pltpu.async_copy(dst_hbm.at[pl.ds(base, 8), :], dbuf, sem_i).wait()
        puts = [
            pltpu.async_copy(ones, acc.at[dbuf.at[i]], sem_s, add=True)
            for i in range(8)
        ]
        for p in puts:
            p.wait()

    plsc.subcore_barrier()
    _dump_acc(acc, deg_hbm)

    base = wid * (N_D // NS)
    lc = jnp.zeros((C,), jnp.float32)
    lanes = lax.iota(jnp.int32, C)
    count_chunks = tuple((o * 512, 512) for o in range(12)) + ((6144, 256),)
    for off, rows in count_chunks:
        c1 = pltpu.async_copy(y_hbm.at[pl.ds(base + off, rows)],
                              ybuf.at[pl.ds(0, rows)], sem_i)
        c2 = pltpu.async_copy(tm_hbm.at[pl.ds(base + off, rows)],
                              tbuf.at[pl.ds(0, rows)], sem_i)
        c1.wait()
        c2.wait()

        def body(i, carry):
            yv = ybuf[pl.ds(i, 16)][0]
            tv = tbuf[pl.ds(i, 16)][0]
            return carry + jnp.where(lanes == yv, tv, 0.0)

        lc = pl.loop(0, rows, init_carry=lc)(body)
    lcbuf[...] = lc
    pltpu.sync_copy(lcbuf, lc_hbm.at[wid])


def _make_dense(teleport):
    if teleport is None:
        out_type = (
            jax.ShapeDtypeStruct((N_D, C), jnp.float32),
            jax.ShapeDtypeStruct((N_D, C), jnp.float32),
            jax.ShapeDtypeStruct((N_D, C), jnp.float32),
        )
    else:
        out_type = (
            jax.ShapeDtypeStruct((N_D, C), jnp.float32),
            jax.ShapeDtypeStruct((N_D, C), jnp.float32),
        )

    @functools.partial(
        pl.kernel,
        out_type=out_type,
        mesh=_mesh,
        compiler_params=_cparams,
        scratch_types=[
            pltpu.VMEM((512, C), jnp.float32),
            pltpu.VMEM((512, C), jnp.float32),
            pltpu.VMEM((512, C), jnp.float32),
            pltpu.VMEM((512, C), jnp.float32),
            pltpu.VMEM((512, C), jnp.float32),
            pltpu.VMEM((512, C), jnp.float32),
            pltpu.VMEM((528,), jnp.int32),
            pltpu.VMEM((528,), jnp.float32),
            pltpu.VMEM((C,), jnp.float32),
            pltpu.SemaphoreType.DMA,
            pltpu.SemaphoreType.DMA,
        ],
    )
    def dense_k(*args):
        if teleport is None:
            (deg_hbm, y_hbm, tm_hbm, lc_hbm, dinv_hbm, puc_hbm, u0_hbm,
             p0, p1, _hh, _pp, outa, outb, ybuf, tbuf, lcv,
             sem_i, sem_o) = args
        elif teleport:
            (part_hbm, h_hbm, dinv_hbm, puc_hbm, hn_hbm, un_hbm,
             p0, p1, hh, pp, outa, outb, _yb, _tb, _lcv,
             sem_i, sem_o) = args
        else:
            (part_hbm, h_hbm, dinv_hbm, hn_hbm, un_hbm,
             p0, p1, hh, pp, outa, outb, _yb, _tb, _lcv,
             sem_i, sem_o) = args
        wid = _wid()
        base = wid * RPW_D
        lanes = lax.iota(jnp.int32, C)
        if teleport is None:
            pltpu.sync_copy(lc_hbm, lcv)
            lcm = jnp.maximum(lcv[...], 1.0)
        for off, rows in DCHUNKS:
            r0 = base + off
            cps = []
            if teleport is None:
                cps.append(pltpu.async_copy(
                    deg_hbm.at[pl.ds(r0, rows), :],
                    p0.at[pl.ds(0, rows), :], sem_i))
                cps.append(pltpu.async_copy(
                    y_hbm.at[pl.ds(r0, rows)], ybuf.at[pl.ds(0, rows)],
                    sem_i))
                cps.append(pltpu.async_copy(
                    tm_hbm.at[pl.ds(r0, rows)], tbuf.at[pl.ds(0, rows)],
                    sem_i))
            else:
                cps.append(pltpu.async_copy(
                    part_hbm.at[pl.ds(r0, rows), :],
                    p0.at[pl.ds(0, rows), :], sem_i))
                cps.append(pltpu.async_copy(
                    h_hbm.at[pl.ds(r0, rows), :],
                    hh.at[pl.ds(0, rows), :], sem_i))
                cps.append(pltpu.async_copy(
                    dinv_hbm.at[pl.ds(r0, rows), :],
                    pp.at[pl.ds(0, rows), :], sem_i))
                if teleport:
                    cps.append(pltpu.async_copy(
                        puc_hbm.at[pl.ds(r0, rows), :],
                        outb.at[pl.ds(0, rows), :], sem_i))
            for cp in cps:
                cp.wait()

            if teleport is None:
                @pl.loop(0, rows)
                def _(i):
                    deg = p0[i] + 1.0
                    d = _rsqrt(jnp.maximum(deg, 1.0))
                    yv = ybuf[pl.ds(i, 16)][0]
                    tv = tbuf[pl.ds(i, 16)][0]
                    puc = jnp.where(lanes == yv, tv, 0.0) / lcm
                    p0[i] = d
                    p1[i] = puc
                    outa[i] = d * puc
                ocps = [
                    pltpu.async_copy(p0.at[pl.ds(0, rows), :],
                                     dinv_hbm.at[pl.ds(r0, rows), :], sem_o),
                    pltpu.async_copy(p1.at[pl.ds(0, rows), :],
                                     puc_hbm.at[pl.ds(r0, rows), :], sem_o),
                    pltpu.async_copy(outa.at[pl.ds(0, rows), :],
                                     u0_hbm.at[pl.ds(r0, rows), :], sem_o),
                ]
            else:
                @pl.loop(0, rows)
                def _(i):
                    d = pp[i]
                    prop = d * p0[i] + d * d * hh[i]
                    if teleport:
                        hn = (1.0 - ALPHA) * prop + ALPHA * outb[i]
                    else:
                        hn = prop
                    outa[i] = hn
                    hh[i] = d * hn
                ocps = [
                    pltpu.async_copy(outa.at[pl.ds(0, rows), :],
                                     hn_hbm.at[pl.ds(r0, rows), :], sem_o),
                    pltpu.async_copy(hh.at[pl.ds(0, rows), :],
                                     un_hbm.at[pl.ds(r0, rows), :], sem_o),
                ]
            for cp in ocps:
                cp.wait()

    return dense_k


_init_k = _make_dense(None)
_blend_ap = _make_dense(True)
_blend_pr = _make_dense(False)


@functools.partial(
    pl.kernel,
    out_type=(
        jax.ShapeDtypeStruct((N_D, C), jnp.float32),
        jax.ShapeDtypeStruct((N_D // 16, C), jnp.float32),
    ),
    mesh=_mesh,
    compiler_params=_cparams,
    scratch_types=[
        pltpu.VMEM((512, C), jnp.float32),
        pltpu.VMEM((512, C), jnp.float32),
        pltpu.VMEM((512, C), jnp.float32),
        pltpu.VMEM((32, C), jnp.float32),
        pltpu.VMEM((C,), jnp.float32),
        pltpu.SemaphoreType.DMA,
        pltpu.SemaphoreType.DMA,
    ],
)
def _final_k(h_hbm, g_hbm, lc_hbm, out_hbm, pu_hbm,
             hh, gg, oo, pub, lcv, sem_i, sem_o):
    wid = _wid()
    base = wid * RPW_D
    lanes = lax.iota(jnp.int32, C)
    pltpu.sync_copy(lc_hbm, lcv)
    lc = lcv[...]
    pc = lc / jnp.sum(lc)
    for off, rows in DCHUNKS:
        r0 = base + off
        c1 = pltpu.async_copy(h_hbm.at[pl.ds(r0, rows), :],
                              hh.at[pl.ds(0, rows), :], sem_i)
        c2 = pltpu.async_copy(g_hbm.at[pl.ds(r0, rows), :],
                              gg.at[pl.ds(0, rows), :], sem_i)
        c1.wait()
        c2.wait()

        @pl.loop(0, rows // 16)
        def _(gr):
            pu = jnp.zeros((C,), jnp.float32)
            for l in range(16):
                i = gr * 16 + l
                o = (WMIX * hh[i] + (1.0 - WMIX) * gg[i]) * pc
                oo[i] = o
                pu = jnp.where(lanes == l, jnp.sum(o), pu)
            pub[gr] = pu
        ocps = [
            pltpu.async_copy(oo.at[pl.ds(0, rows), :],
                             out_hbm.at[pl.ds(r0, rows), :], sem_o),
            pltpu.async_copy(pub.at[pl.ds(0, rows // 16), :],
                             pu_hbm.at[pl.ds(r0 // 16, rows // 16), :],
                             sem_o),
        ]
        for cp in ocps:
            cp.wait()


@jax.jit
def kernel(x, y, train_mask, edge_index):
    del x
    pad_e = E_PAD - E
    src = jnp.concatenate(
        [edge_index[0], jnp.full((pad_e,), N, jnp.int32)]).reshape(
            IDXROWS, 128)
    dst = jnp.concatenate(
        [edge_index[1], jnp.full((pad_e,), N, jnp.int32)]).reshape(
            IDXROWS, 128)
    pad_n = N_D - N
    y_p = jnp.concatenate([y, jnp.zeros((pad_n,), jnp.int32)])
    tm_p = jnp.concatenate(
        [train_mask.astype(jnp.float32), jnp.zeros((pad_n,), jnp.float32)])

    deg_parts, lc_parts = _count_k(dst, y_p, tm_p)
    lc = lc_parts.sum(0)
    dinv, puc, u0 = _init_k(deg_parts, y_p, tm_p, lc)

    h, u = puc, u0
    for _ in range(K_AP):
        part = _scatter_k(u, src, dst)
        h, u = _blend_ap(part, h, dinv, puc)
    g, u = puc, u0
    for _ in range(K_PR):
        part = _scatter_k(u, src, dst)
        g, u = _blend_pr(part, g, dinv)

    out16, pu = _final_k(h, g, lc)
    out = out16[:N, :NUM_CLASSES]
    p_u = pu.reshape(-1)[:N]
    p_c = lc[:NUM_CLASSES] / lc.sum()
    return (out, p_u, p_c)


# speedup vs baseline: 1.8756x; 1.5680x over previous
import functools

import jax
import jax.numpy as jnp
from jax import lax
from jax.experimental import pallas as pl
from jax.experimental.pallas import tpu as pltpu
from jax.experimental.pallas import tpu_sc as plsc

N = 100000
E = 3200000
C = 16
NUM_CLASSES = 10
NC = 2
NS = 16
NW = NC * NS
K_AP = 10
K_PR = 10
ALPHA = 0.1
WMIX = 0.9

N_D = 102400
CH = 512
CR = CH // 128
NCHUNK = 196
EPW = CH * NCHUNK
NCHUNK_F = EPW // 1024
E_PAD = EPW * NW
IDXROWS = E_PAD // 128
RPW_D = N_D // NW
DCHUNKS = ((0, 512), (512, 512), (1024, 512), (1536, 512),
           (2048, 512), (2560, 512), (3072, 128))
N_ACC = 100096
SL = N_ACC // NS
ZCH = SL // 16

_mesh = plsc.VectorSubcoreMesh(core_axis_name="c", subcore_axis_name="s")
_mesh1 = plsc.VectorSubcoreMesh(core_axis_name="c", subcore_axis_name="s",
                                num_cores=1)
_cparams = pltpu.CompilerParams(use_tc_tiling_on_sc=False,
                                needs_layout_passes=False)


def _wid():
    return lax.axis_index("c") * NS + lax.axis_index("s")


def _rsqrt(x):
    i = lax.bitcast_convert_type(x, jnp.int32)
    i = jnp.int32(0x5F3759DF) - lax.shift_right_arithmetic(i, 1)
    y = lax.bitcast_convert_type(i, jnp.float32)
    for _ in range(4):
        y = y * (1.5 - 0.5 * x * y * y)
    return y


def _zero_acc(acc, zbuf):
    zrow = jnp.zeros((C,), jnp.float32)

    @pl.loop(0, ZCH)
    def _(i):
        zbuf[i] = zrow

    sid = lax.axis_index("s")
    for j in range(16):
        pltpu.sync_copy(zbuf.at[pl.ds(0, ZCH), :],
                        acc.at[pl.ds(sid * SL + j * ZCH, ZCH), :])


def _dump_acc(acc, part_hbm):
    cid = lax.axis_index("c")
    sid = lax.axis_index("s")
    pltpu.sync_copy(acc.at[pl.ds(sid * SL, SL), :],
                    part_hbm.at[cid, pl.ds(sid * SL, SL), :])


@functools.partial(
    pl.kernel,
    out_type=jax.ShapeDtypeStruct((NC, N_D, C), jnp.float32),
    mesh=_mesh,
    compiler_params=_cparams,
    scratch_types=[
        pltpu.VMEM((CR, 128), jnp.int32),
        pltpu.VMEM((CR, 128), jnp.int32),
        pltpu.VMEM((CR, 128), jnp.int32),
        pltpu.VMEM((CR, 128), jnp.int32),
        pltpu.VMEM((CH, C), jnp.float32),
        pltpu.VMEM((CH, C), jnp.float32),
        pltpu.VMEM_SHARED((N_ACC, C), jnp.float32),
        pltpu.SemaphoreType.DMA,
        pltpu.SemaphoreType.DMA,
        pltpu.SemaphoreType.DMA,
    ],
)
def _scatter_k(u_hbm, src_hbm, dst_hbm, part_hbm,
               sb_a, db_a, sb_b, db_b, rows_a, rows_b, acc,
               sem_i, sem_g, sem_s):
    wid = _wid()
    base0 = wid * (EPW // 128)

    def load_idx(ch, sb, db):
        c1 = pltpu.async_copy(src_hbm.at[pl.ds(base0 + ch * CR, CR), :],
                              sb, sem_i)
        c2 = pltpu.async_copy(dst_hbm.at[pl.ds(base0 + ch * CR, CR), :],
                              db, sem_i)
        c1.wait()
        c2.wait()

    def gathers(sb, rows):
        return [pltpu.make_async_copy(u_hbm.at[sb.at[i]],
                                      rows.at[pl.ds(i * 128, 128), :], sem_g)
                for i in range(CR)]

    def scatters(db, rows):
        return [pltpu.make_async_copy(rows.at[pl.ds(i * 128, 128), :],
                                      acc.at[db.at[i]], sem_s)
                for i in range(CR)]

    _zero_acc(acc, rows_a)
    zrow = jnp.zeros((C,), jnp.float32)

    @pl.loop(0, CH)
    def _(i):
        rows_b[i] = zrow

    plsc.subcore_barrier()

    load_idx(0, sb_b, db_b)
    for p in scatters(db_b, rows_b):
        p.start(add=True)
    load_idx(0, sb_a, db_a)
    for g in gathers(sb_a, rows_a):
        g.start()

    @pl.loop(0, NCHUNK, step=2)
    def _(ch):
        for p in scatters(db_b, rows_b):
            p.wait()
        load_idx(ch + 1, sb_b, db_b)
        for g in gathers(sb_a, rows_a):
            g.wait()
        for p in scatters(db_a, rows_a):
            p.start(add=True)
        for g in gathers(sb_b, rows_b):
            g.start()
        for p in scatters(db_a, rows_a):
            p.wait()
        c2 = jnp.minimum(ch + 2, NCHUNK - 1)
        load_idx(c2, sb_a, db_a)
        for g in gathers(sb_b, rows_b):
            g.wait()
        for p in scatters(db_b, rows_b):
            p.start(add=True)
        for g in gathers(sb_a, rows_a):
            g.start()

    for g in gathers(sb_a, rows_a):
        g.wait()
    for p in scatters(db_b, rows_b):
        p.wait()

    plsc.subcore_barrier()
    _dump_acc(acc, part_hbm)


@functools.partial(
    pl.kernel,
    out_type=(
        jax.ShapeDtypeStruct((NC, N_D, C), jnp.float32),
        jax.ShapeDtypeStruct((NW, C), jnp.float32),
    ),
    mesh=_mesh,
    compiler_params=_cparams,
    scratch_types=[
        pltpu.VMEM((8, 128), jnp.int32),
        pltpu.VMEM((128, C), jnp.float32),
        pltpu.VMEM((ZCH, C), jnp.float32),
        pltpu.VMEM((528,), jnp.int32),
        pltpu.VMEM((528,), jnp.float32),
        pltpu.VMEM((C,), jnp.float32),
        pltpu.VMEM_SHARED((N_ACC, C), jnp.float32),
        pltpu.SemaphoreType.DMA,
        pltpu.SemaphoreType.DMA,
    ],
)
def _count_k(dst_hbm, y_hbm, tm_hbm, deg_hbm, lc_hbm,
             dbuf, ones, zbuf, ybuf, tbuf, lcbuf, acc, sem_i, sem_s):
    wid = _wid()
    one_row = jnp.ones((C,), jnp.float32)

    @pl.loop(0, 128)
    def _(i):
        ones[i] = one_row

    _zero_acc(acc, zbuf)
    plsc.subcore_barrier()

    @pl.loop(0, NCHUNK_F)
    def _(ch):
        base = wid * (EPW // 128) + ch * 8
        pltpu.async_copy(dst_hbm.at[pl.ds(base, 8), :], dbuf, sem_i).wait()
        puts = [
            pltpu.async_copy(ones, acc.at[dbuf.at[i]], sem_s, add=True)
            for i in range(8)
        ]
        for p in puts:
            p.wait()

    plsc.subcore_barrier()
    _dump_acc(acc, deg_hbm)

    base = wid * RPW_D
    lc = jnp.zeros((C,), jnp.float32)
    lanes = lax.iota(jnp.int32, C)
    for off, rows in DCHUNKS:
        c1 = pltpu.async_copy(y_hbm.at[pl.ds(base + off, rows)],
                              ybuf.at[pl.ds(0, rows)], sem_i)
        c2 = pltpu.async_copy(tm_hbm.at[pl.ds(base + off, rows)],
                              tbuf.at[pl.ds(0, rows)], sem_i)
        c1.wait()
        c2.wait()

        def body(i, carry):
            yv = ybuf[pl.ds(i, 16)][0]
            tv = tbuf[pl.ds(i, 16)][0]
            return carry + jnp.where(lanes == yv, tv, 0.0)

        lc = pl.loop(0, rows, init_carry=lc)(body)
    lcbuf[...] = lc
    pltpu.sync_copy(lcbuf, lc_hbm.at[wid])


def _make_dense(teleport):
    if teleport is None:
        out_type = (
            jax.ShapeDtypeStruct((N_D, C), jnp.float32),
            jax.ShapeDtypeStruct((N_D, C), jnp.float32),
            jax.ShapeDtypeStruct((N_D, C), jnp.float32),
        )
    else:
        out_type = (
            jax.ShapeDtypeStruct((N_D, C), jnp.float32),
            jax.ShapeDtypeStruct((N_D, C), jnp.float32),
        )

    @functools.partial(
        pl.kernel,
        out_type=out_type,
        mesh=_mesh,
        compiler_params=_cparams,
        scratch_types=[
            pltpu.VMEM((512, C), jnp.float32),
            pltpu.VMEM((512, C), jnp.float32),
            pltpu.VMEM((512, C), jnp.float32),
            pltpu.VMEM((512, C), jnp.float32),
            pltpu.VMEM((512, C), jnp.float32),
            pltpu.VMEM((512, C), jnp.float32),
            pltpu.VMEM((528,), jnp.int32),
            pltpu.VMEM((528,), jnp.float32),
            pltpu.VMEM((C,), jnp.float32),
            pltpu.SemaphoreType.DMA,
            pltpu.SemaphoreType.DMA,
        ],
    )
    def dense_k(*args):
        if teleport is None:
            (deg_hbm, y_hbm, tm_hbm, lc_hbm, dinv_hbm, puc_hbm, u0_hbm,
             p0, p1, _hh, _pp, outa, outb, ybuf, tbuf, lcv,
             sem_i, sem_o) = args
        elif teleport:
            (part_hbm, h_hbm, dinv_hbm, puc_hbm, hn_hbm, un_hbm,
             p0, p1, hh, pp, outa, outb, _yb, _tb, _lcv,
             sem_i, sem_o) = args
        else:
            (part_hbm, h_hbm, dinv_hbm, hn_hbm, un_hbm,
             p0, p1, hh, pp, outa, outb, _yb, _tb, _lcv,
             sem_i, sem_o) = args
        wid = _wid()
        base = wid * RPW_D
        lanes = lax.iota(jnp.int32, C)
        if teleport is None:
            pltpu.sync_copy(lc_hbm, lcv)
            lcm = jnp.maximum(lcv[...], 1.0)
        for off, rows in DCHUNKS:
            r0 = base + off
            cps = []
            if teleport is None:
                cps.append(pltpu.async_copy(
                    deg_hbm.at[0, pl.ds(r0, rows), :],
                    p0.at[pl.ds(0, rows), :], sem_i))
                cps.append(pltpu.async_copy(
                    deg_hbm.at[1, pl.ds(r0, rows), :],
                    p1.at[pl.ds(0, rows), :], sem_i))
                cps.append(pltpu.async_copy(
                    y_hbm.at[pl.ds(r0, rows)], ybuf.at[pl.ds(0, rows)],
                    sem_i))
                cps.append(pltpu.async_copy(
                    tm_hbm.at[pl.ds(r0, rows)], tbuf.at[pl.ds(0, rows)],
                    sem_i))
            else:
                cps.append(pltpu.async_copy(
                    part_hbm.at[0, pl.ds(r0, rows), :],
                    p0.at[pl.ds(0, rows), :], sem_i))
                cps.append(pltpu.async_copy(
                    part_hbm.at[1, pl.ds(r0, rows), :],
                    p1.at[pl.ds(0, rows), :], sem_i))
                cps.append(pltpu.async_copy(
                    h_hbm.at[pl.ds(r0, rows), :],
                    hh.at[pl.ds(0, rows), :], sem_i))
                cps.append(pltpu.async_copy(
                    dinv_hbm.at[pl.ds(r0, rows), :],
                    pp.at[pl.ds(0, rows), :], sem_i))
                if teleport:
                    cps.append(pltpu.async_copy(
                        puc_hbm.at[pl.ds(r0, rows), :],
                        outb.at[pl.ds(0, rows), :], sem_i))
            for cp in cps:
                cp.wait()

            if teleport is None:
                @pl.loop(0, rows)
                def _(i):
                    deg = p0[i] + p1[i] + 1.0
                    d = _rsqrt(jnp.maximum(deg, 1.0))
                    yv = ybuf[pl.ds(i, 16)][0]
                    tv = tbuf[pl.ds(i, 16)][0]
                    puc = jnp.where(lanes == yv, tv, 0.0) / lcm
                    p0[i] = d
                    p1[i] = puc
                    outa[i] = d * puc
                ocps = [
                    pltpu.async_copy(p0.at[pl.ds(0, rows), :],
                                     dinv_hbm.at[pl.ds(r0, rows), :], sem_o),
                    pltpu.async_copy(p1.at[pl.ds(0, rows), :],
                                     puc_hbm.at[pl.ds(r0, rows), :], sem_o),
                    pltpu.async_copy(outa.at[pl.ds(0, rows), :],
                                     u0_hbm.at[pl.ds(r0, rows), :], sem_o),
                ]
            else:
                @pl.loop(0, rows)
                def _(i):
                    d = pp[i]
                    prop = d * (p0[i] + p1[i]) + d * d * hh[i]
                    if teleport:
                        hn = (1.0 - ALPHA) * prop + ALPHA * outb[i]
                    else:
                        hn = prop
                    outa[i] = hn
                    hh[i] = d * hn
                ocps = [
                    pltpu.async_copy(outa.at[pl.ds(0, rows), :],
                                     hn_hbm.at[pl.ds(r0, rows), :], sem_o),
                    pltpu.async_copy(hh.at[pl.ds(0, rows), :],
                                     un_hbm.at[pl.ds(r0, rows), :], sem_o),
                ]
            for cp in ocps:
                cp.wait()

    return dense_k


_init_k = _make_dense(None)
_blend_ap = _make_dense(True)
_blend_pr = _make_dense(False)


@functools.partial(
    pl.kernel,
    out_type=(
        jax.ShapeDtypeStruct((N_D, C), jnp.float32),
        jax.ShapeDtypeStruct((N_D // 16, C), jnp.float32),
    ),
    mesh=_mesh,
    compiler_params=_cparams,
    scratch_types=[
        pltpu.VMEM((512, C), jnp.float32),
        pltpu.VMEM((512, C), jnp.float32),
        pltpu.VMEM((512, C), jnp.float32),
        pltpu.VMEM((32, C), jnp.float32),
        pltpu.VMEM((C,), jnp.float32),
        pltpu.SemaphoreType.DMA,
        pltpu.SemaphoreType.DMA,
    ],
)
def _final_k(h_hbm, g_hbm, lc_hbm, out_hbm, pu_hbm,
             hh, gg, oo, pub, lcv, sem_i, sem_o):
    wid = _wid()
    base = wid * RPW_D
    lanes = lax.iota(jnp.int32, C)
    pltpu.sync_copy(lc_hbm, lcv)
    lc = lcv[...]
    pc = lc / jnp.sum(lc)
    for off, rows in DCHUNKS:
        r0 = base + off
        c1 = pltpu.async_copy(h_hbm.at[pl.ds(r0, rows), :],
                              hh.at[pl.ds(0, rows), :], sem_i)
        c2 = pltpu.async_copy(g_hbm.at[pl.ds(r0, rows), :],
                              gg.at[pl.ds(0, rows), :], sem_i)
        c1.wait()
        c2.wait()

        @pl.loop(0, rows // 16)
        def _(gr):
            pu = jnp.zeros((C,), jnp.float32)
            for l in range(16):
                i = gr * 16 + l
                o = (WMIX * hh[i] + (1.0 - WMIX) * gg[i]) * pc
                oo[i] = o
                pu = jnp.where(lanes == l, jnp.sum(o), pu)
            pub[gr] = pu
        ocps = [
            pltpu.async_copy(oo.at[pl.ds(0, rows), :],
                             out_hbm.at[pl.ds(r0, rows), :], sem_o),
            pltpu.async_copy(pub.at[pl.ds(0, rows // 16), :],
                             pu_hbm.at[pl.ds(r0 // 16, rows // 16), :],
                             sem_o),
        ]
        for cp in ocps:
            cp.wait()


@jax.jit
def kernel(x, y, train_mask, edge_index):
    del x
    pad_e = E_PAD - E
    src = jnp.concatenate(
        [edge_index[0], jnp.full((pad_e,), N, jnp.int32)]).reshape(
            IDXROWS, 128)
    dst = jnp.concatenate(
        [edge_index[1], jnp.full((pad_e,), N, jnp.int32)]).reshape(
            IDXROWS, 128)
    pad_n = N_D - N
    y_p = jnp.concatenate([y, jnp.zeros((pad_n,), jnp.int32)])
    tm_p = jnp.concatenate(
        [train_mask.astype(jnp.float32), jnp.zeros((pad_n,), jnp.float32)])

    deg_parts, lc_parts = _count_k(dst, y_p, tm_p)
    lc = lc_parts.sum(0)
    dinv, puc, u0 = _init_k(deg_parts, y_p, tm_p, lc)

    h, u = puc, u0
    for _ in range(K_AP):
        part = _scatter_k(u, src, dst)
        h, u = _blend_ap(part, h, dinv, puc)
    g, u = puc, u0
    for _ in range(K_PR):
        part = _scatter_k(u, src, dst)
        g, u = _blend_pr(part, g, dinv)

    out16, pu = _final_k(h, g, lc)
    out = out16[:N, :NUM_CLASSES]
    p_u = pu.reshape(-1)[:N]
    p_c = lc[:NUM_CLASSES] / lc.sum()
    return (out, p_u, p_c)
